# Initial kernel scaffold; baseline (speedup 1.0000x reference)
import jax
import jax.numpy as jnp
from jax.experimental import pallas as pl


def kernel(x, edge_index, batch, W1, b1, W2, b2):
    raise NotImplementedError("write your pallas kernel here")


# speedup vs baseline: 27.7027x; 27.7027x over previous
import functools

import jax
import jax.numpy as jnp
from jax import lax
from jax.experimental import pallas as pl
from jax.experimental.pallas import tpu as pltpu
from jax.experimental.pallas import tpu_sc as plsc

N = 50000
E = 800000
DIN = 78
DPAD = 80
DH = 156
DE = 32
G = 256

NC = 2
NS = 16
LANES = 16

CH = 5
EB = 1250
NSTR = 10
IPS = 125
NBLK = E // EB
NP8 = 50048
ROWS_PER_TILE = NP8 // NS
ZROWS = IPS

def _sc_mesh():
    return plsc.VectorSubcoreMesh(core_axis_name="c", subcore_axis_name="s",
                                  num_cores=NC, num_subcores=NS)


def _zero_spmem(spmem, zbuf, s):
    base = s * ROWS_PER_TILE
    pltpu.sync_copy(zbuf.at[pl.ds(0, 1248)], spmem.at[pl.ds(base, 1248)])
    pltpu.sync_copy(zbuf.at[pl.ds(0, 1248)], spmem.at[pl.ds(base + 1248, 1248)])
    pltpu.sync_copy(zbuf.at[pl.ds(0, 632)], spmem.at[pl.ds(base + 2496, 632)])


def _fill_rows(buf, val):
    @pl.loop(0, buf.shape[0])
    def _(i):
        buf[i] = jnp.full((LANES,), val, jnp.float32)


def _deg_body(dstb, degp, spmem, obuf, zbuf, didx0, didx1, isem0, isem1,
              ssem0, ssem1):
    c = lax.axis_index("c")
    s = lax.axis_index("s")
    wid = c * NS + s
    didx = (didx0, didx1)
    isem = (isem0, isem1)
    ssem = (ssem0, ssem1)

    _fill_rows(obuf, 1.0)
    _fill_rows(zbuf, 0.0)
    _zero_spmem(spmem, zbuf, s)
    plsc.subcore_barrier()

    per_tile = NBLK // (NC * NS)
    b0 = wid * per_tile

    def fire_scatters(k):
        @pl.loop(0, NSTR)
        def _(j):
            pltpu.async_copy(obuf, spmem.at[didx[k].at[j]], ssem[k], add=True)

    def drain_scatters(k):
        @pl.loop(0, NSTR)
        def _(j):
            pltpu.make_async_copy(obuf, spmem.at[didx[k].at[j]], ssem[k]).wait()

    pltpu.async_copy(dstb.at[b0], didx[0], isem[0])

    @pl.loop(0, per_tile, step=2)
    def _(bb):
        for k in (0, 1):
            b = bb + k
            blk = b0 + b
            pltpu.make_async_copy(dstb.at[blk], didx[k], isem[k]).wait()
            fire_scatters(k)
            if k == 0:
                @pl.when(bb > 0)
                def _():
                    drain_scatters(1)
                pltpu.async_copy(dstb.at[blk + 1], didx[1], isem[1])
            else:
                drain_scatters(0)

                @pl.when(bb + 2 < per_tile)
                def _():
                    pltpu.async_copy(dstb.at[blk + 1], didx[0], isem[0])

    drain_scatters(1)
    plsc.subcore_barrier()
    pltpu.sync_copy(spmem.at[pl.ds(s * ROWS_PER_TILE, ROWS_PER_TILE)],
                    degp.at[c].at[pl.ds(s * ROWS_PER_TILE, ROWS_PER_TILE)])


_SC_PARAMS = pltpu.CompilerParams(use_tc_tiling_on_sc=False)


@jax.jit
def _deg_kernel(dstb):
    return pl.kernel(
        _deg_body,
        compiler_params=_SC_PARAMS,
        out_type=jax.ShapeDtypeStruct((NC, NP8, LANES), jnp.float32),
        mesh=_sc_mesh(),
        scratch_types=[
            pltpu.VMEM_SHARED((NP8, LANES), jnp.float32),
            pltpu.VMEM((ZROWS, LANES), jnp.float32),
            pltpu.VMEM((1248, LANES), jnp.float32),
            pltpu.VMEM((NSTR, IPS), jnp.int32),
            pltpu.VMEM((NSTR, IPS), jnp.int32),
            pltpu.SemaphoreType.DMA,
            pltpu.SemaphoreType.DMA,
            pltpu.SemaphoreType.DMA,
            pltpu.SemaphoreType.DMA,
        ],
    )(dstb)


def _agg_body(ytab, srcb, dstb, agg, spmem, sidx0, sidx1, didx0, didx1,
              rows0, rows1, isem0, isem1, gsem0, gsem1, ssem0, ssem1):
    c = lax.axis_index("c")
    s = lax.axis_index("s")
    sidx = (sidx0, sidx1)
    didx = (didx0, didx1)
    rows = (rows0, rows1)
    isem = (isem0, isem1)
    gsem = (gsem0, gsem1)
    ssem = (ssem0, ssem1)

    is0 = c == 0
    tabs = (jnp.where(is0, 0, 3), jnp.where(is0, 1, 4), jnp.where(is0, 2, 2))
    slots = (jnp.where(is0, 0, 3), jnp.where(is0, 1, 4), jnp.where(is0, 2, 5))
    bases = (0, 0, jnp.where(is0, 0, NBLK // 2))
    per_tiles = (NBLK // NS, NBLK // NS, (NBLK // 2) // NS)

    for tab, slot, base, per_tile in zip(tabs, slots, bases, per_tiles):
        ytab_t = ytab.at[tab]
        _fill_rows(rows0, 0.0)
        _zero_spmem(spmem, rows0, s)
        plsc.subcore_barrier()
        tile_b0 = base + s * per_tile

        def fetch_idx(blk, k):
            pltpu.async_copy(srcb.at[blk], sidx[k], isem[k])
            pltpu.async_copy(dstb.at[blk], didx[k], isem[k])

        def wait_idx(blk, k):
            pltpu.make_async_copy(srcb.at[blk], sidx[k], isem[k]).wait()
            pltpu.make_async_copy(dstb.at[blk], didx[k], isem[k]).wait()

        def fire_gathers(k):
            @pl.loop(0, NSTR)
            def _(j):
                pltpu.async_copy(ytab_t.at[sidx[k].at[j]],
                                 rows[k].at[pl.ds(j * IPS, IPS)], gsem[k])

        def drain_gathers(k):
            @pl.loop(0, NSTR)
            def _(j):
                pltpu.make_async_copy(ytab_t.at[sidx[k].at[j]],
                                      rows[k].at[pl.ds(j * IPS, IPS)],
                                      gsem[k]).wait()

        def fire_scatters(k):
            @pl.loop(0, NSTR)
            def _(j):
                pltpu.async_copy(rows[k].at[pl.ds(j * IPS, IPS)],
                                 spmem.at[didx[k].at[j]], ssem[k], add=True)

        def drain_scatters(k):
            @pl.loop(0, NSTR)
            def _(j):
                pltpu.make_async_copy(rows[k].at[pl.ds(j * IPS, IPS)],
                                      spmem.at[didx[k].at[j]], ssem[k]).wait()

        fetch_idx(tile_b0, 0)

        @pl.loop(0, per_tile, step=2)
        def _(bb):
            for k in (0, 1):
                b = bb + k
                blk = tile_b0 + b
                wait_idx(blk, k)
                fire_gathers(k)
                if k == 0:
                    @pl.when(bb > 0)
                    def _():
                        drain_scatters(1)
                    fetch_idx(blk + 1, 1)
                else:
                    drain_scatters(0)

                    @pl.when(bb + 2 < per_tile)
                    def _():
                        fetch_idx(blk + 1, 0)
                drain_gathers(k)
                fire_scatters(k)

        drain_scatters(1)
        plsc.subcore_barrier()
        pltpu.sync_copy(spmem.at[pl.ds(s * ROWS_PER_TILE, ROWS_PER_TILE)],
                        agg.at[slot].at[pl.ds(s * ROWS_PER_TILE, ROWS_PER_TILE)])
        plsc.subcore_barrier()


@jax.jit
def _agg_kernel(ytab, srcb, dstb):
    return pl.kernel(
        _agg_body,
        compiler_params=_SC_PARAMS,
        out_type=jax.ShapeDtypeStruct((6, NP8, LANES), jnp.float32),
        mesh=_sc_mesh(),
        scratch_types=[
            pltpu.VMEM_SHARED((NP8, LANES), jnp.float32),
            pltpu.VMEM((NSTR, IPS), jnp.int32),
            pltpu.VMEM((NSTR, IPS), jnp.int32),
            pltpu.VMEM((NSTR, IPS), jnp.int32),
            pltpu.VMEM((NSTR, IPS), jnp.int32),
            pltpu.VMEM((EB, LANES), jnp.float32),
            pltpu.VMEM((EB, LANES), jnp.float32),
            pltpu.SemaphoreType.DMA,
            pltpu.SemaphoreType.DMA,
            pltpu.SemaphoreType.DMA,
            pltpu.SemaphoreType.DMA,
            pltpu.SemaphoreType.DMA,
            pltpu.SemaphoreType.DMA,
        ],
    )(ytab, srcb, dstb)


RB = 2000
NBB = N // RB


def _prep_body(deg_ref, x_ref, y_ref, dinv_ref):
    deg = deg_ref[0, :, 0:1] + deg_ref[1, :, 0:1] + 1.0
    dinv = lax.rsqrt(deg)
    dinv_ref[...] = dinv
    y = x_ref[...] * dinv
    ypad = jnp.concatenate([y, jnp.zeros((RB, DPAD - DIN), jnp.float32)], axis=1)
    for t in range(CH):
        y_ref[t] = ypad[:, t * LANES:(t + 1) * LANES]


@jax.jit
def _prep_kernel(degp, x):
    return pl.pallas_call(
        _prep_body,
        grid=(NBB,),
        in_specs=[
            pl.BlockSpec((NC, RB, LANES), lambda i: (0, i, 0)),
            pl.BlockSpec((RB, DIN), lambda i: (i, 0)),
        ],
        out_specs=[
            pl.BlockSpec((CH, RB, LANES), lambda i: (0, i, 0)),
            pl.BlockSpec((RB, 1), lambda i: (i, 0)),
        ],
        out_shape=[
            jax.ShapeDtypeStruct((CH, N, LANES), jnp.float32),
            jax.ShapeDtypeStruct((N, 1), jnp.float32),
        ],
    )(degp, x)


RD = 1000
NBD = N // RD


def _head_body(gf_ref, gl_ref, agg_ref, y_ref, dinv_ref, batch_ref,
               w1_ref, b1_ref, w2_ref, b2_ref, out_ref, pooled):
    i = pl.program_id(0)

    @pl.when(i == 0)
    def _():
        pooled[...] = jnp.zeros_like(pooled)

    parts = [agg_ref[0], agg_ref[1], agg_ref[2] + agg_ref[5],
             agg_ref[3], agg_ref[4]]
    pre = jnp.concatenate([p + y_ref[t] for t, p in enumerate(parts)], axis=1)
    pre = pre * dinv_ref[...]
    h = jnp.dot(pre, w1_ref[...], preferred_element_type=jnp.float32,
                precision=lax.Precision.HIGHEST)
    h = jnp.maximum(h + b1_ref[...], 0.0)

    bcol = batch_ref[...]
    g0 = gf_ref[i]
    g1 = gl_ref[i]

    def upd(kk, _):
        g = g0 + kk
        m = bcol == g
        colmax = jnp.max(jnp.where(m, h, 0.0), axis=0)[None, None, :]
        pooled[pl.ds(g, 1)] = jnp.maximum(pooled[pl.ds(g, 1)], colmax)
        return 0

    lax.fori_loop(0, g1 - g0 + 1, upd, 0)

    @pl.when(i == NBD - 1)
    def _():
        pool2d = pooled[...][:, 0, :]
        o = jnp.dot(pool2d, w2_ref[...], preferred_element_type=jnp.float32,
                    precision=lax.Precision.HIGHEST)
        out_ref[...] = jnp.maximum(o + b2_ref[...], 0.0)


@jax.jit
def _head_kernel(gf, gl, agg, ytab, dinv, batchcol, w1p, b1r, w2, b2r):
    grid_spec = pltpu.PrefetchScalarGridSpec(
        num_scalar_prefetch=2,
        grid=(NBD,),
        in_specs=[
            pl.BlockSpec((6, RD, LANES), lambda i, *_: (0, i, 0)),
            pl.BlockSpec((CH, RD, LANES), lambda i, *_: (0, i, 0)),
            pl.BlockSpec((RD, 1), lambda i, *_: (i, 0)),
            pl.BlockSpec((RD, 1), lambda i, *_: (i, 0)),
            pl.BlockSpec((DPAD, DH), lambda i, *_: (0, 0)),
            pl.BlockSpec((1, DH), lambda i, *_: (0, 0)),
            pl.BlockSpec((DH, DE), lambda i, *_: (0, 0)),
            pl.BlockSpec((1, DE), lambda i, *_: (0, 0)),
        ],
        out_specs=pl.BlockSpec((G, DE), lambda i, *_: (0, 0)),
        scratch_shapes=[pltpu.VMEM((G, 1, DH), jnp.float32)],
    )
    return pl.pallas_call(
        _head_body,
        grid_spec=grid_spec,
        out_shape=jax.ShapeDtypeStruct((G, DE), jnp.float32),
    )(gf, gl, agg, ytab, dinv, batchcol, w1p, b1r, w2, b2r)


def kernel(x, edge_index, batch, W1, b1, W2, b2):
    src = edge_index[0].astype(jnp.int32).reshape(NBLK, NSTR, IPS)
    dst = edge_index[1].astype(jnp.int32).reshape(NBLK, NSTR, IPS)
    batch32 = batch.astype(jnp.int32)
    batchcol = batch32.reshape(N, 1)
    gf = batch32[::RD]
    gl = batch32[RD - 1::RD]
    w1p = jnp.concatenate([W1, jnp.zeros((DPAD - DIN, DH), jnp.float32)], axis=0)

    degp = _deg_kernel(dst)
    ytab, dinv = _prep_kernel(degp, x)
    agg = _agg_kernel(ytab, src, dst)
    return _head_kernel(gf, gl, agg, ytab, dinv, batchcol, w1p,
                        b1.reshape(1, DH), W2, b2.reshape(1, DE))


# speedup vs baseline: 35.1477x; 1.2687x over previous
import functools

import jax
import jax.numpy as jnp
from jax import lax
from jax.experimental import pallas as pl
from jax.experimental.pallas import tpu as pltpu
from jax.experimental.pallas import tpu_sc as plsc

N = 50000
E = 800000
DIN = 78
DPAD = 80
DH = 156
DE = 32
G = 256

NC = 2
NS = 16
LANES = 16

CH = 5
EB = 1250
NSTR = 10
IPS = 125
NBLK = E // EB
NP8 = 50048
ROWS_PER_TILE = NP8 // NS
ZROWS = IPS

def _sc_mesh():
    return plsc.VectorSubcoreMesh(core_axis_name="c", subcore_axis_name="s",
                                  num_cores=NC, num_subcores=NS)


def _zero_spmem(spmem, zbuf, s):
    base = s * ROWS_PER_TILE
    pltpu.sync_copy(zbuf.at[pl.ds(0, 1248)], spmem.at[pl.ds(base, 1248)])
    pltpu.sync_copy(zbuf.at[pl.ds(0, 1248)], spmem.at[pl.ds(base + 1248, 1248)])
    pltpu.sync_copy(zbuf.at[pl.ds(0, 632)], spmem.at[pl.ds(base + 2496, 632)])


def _fill_rows(buf, val):
    @pl.loop(0, buf.shape[0])
    def _(i):
        buf[i] = jnp.full((LANES,), val, jnp.float32)


def _deg_body(dstb, degp, spmem, obuf, zbuf, didx0, didx1, isem0, isem1,
              ssem0, ssem1):
    c = lax.axis_index("c")
    s = lax.axis_index("s")
    wid = c * NS + s
    didx = (didx0, didx1)
    isem = (isem0, isem1)
    ssem = (ssem0, ssem1)

    _fill_rows(obuf, 1.0)
    _fill_rows(zbuf, 0.0)
    _zero_spmem(spmem, zbuf, s)
    plsc.subcore_barrier()

    per_tile = NBLK // (NC * NS)
    b0 = wid * per_tile

    def fire_scatters(k):
        @pl.loop(0, NSTR)
        def _(j):
            pltpu.async_copy(obuf, spmem.at[didx[k].at[j]], ssem[k], add=True)

    def drain_scatters(k):
        @pl.loop(0, NSTR)
        def _(j):
            pltpu.make_async_copy(obuf, spmem.at[didx[k].at[j]], ssem[k]).wait()

    pltpu.async_copy(dstb.at[b0], didx[0], isem[0])

    @pl.loop(0, per_tile, step=2)
    def _(bb):
        for k in (0, 1):
            b = bb + k
            blk = b0 + b
            pltpu.make_async_copy(dstb.at[blk], didx[k], isem[k]).wait()
            fire_scatters(k)
            if k == 0:
                @pl.when(bb > 0)
                def _():
                    drain_scatters(1)
                pltpu.async_copy(dstb.at[blk + 1], didx[1], isem[1])
            else:
                drain_scatters(0)

                @pl.when(bb + 2 < per_tile)
                def _():
                    pltpu.async_copy(dstb.at[blk + 1], didx[0], isem[0])

    drain_scatters(1)
    plsc.subcore_barrier()
    pltpu.sync_copy(spmem.at[pl.ds(s * ROWS_PER_TILE, ROWS_PER_TILE)],
                    degp.at[c].at[pl.ds(s * ROWS_PER_TILE, ROWS_PER_TILE)])


_SC_PARAMS = pltpu.CompilerParams(use_tc_tiling_on_sc=False)


@jax.jit
def _deg_kernel(dstb):
    return pl.kernel(
        _deg_body,
        compiler_params=_SC_PARAMS,
        out_type=jax.ShapeDtypeStruct((NC, NP8, LANES), jnp.float32),
        mesh=_sc_mesh(),
        scratch_types=[
            pltpu.VMEM_SHARED((NP8, LANES), jnp.float32),
            pltpu.VMEM((ZROWS, LANES), jnp.float32),
            pltpu.VMEM((1248, LANES), jnp.float32),
            pltpu.VMEM((NSTR, IPS), jnp.int32),
            pltpu.VMEM((NSTR, IPS), jnp.int32),
            pltpu.SemaphoreType.DMA,
            pltpu.SemaphoreType.DMA,
            pltpu.SemaphoreType.DMA,
            pltpu.SemaphoreType.DMA,
        ],
    )(dstb)


def _agg_body(ytab, srcb, dstb, agg, spmem, sidx0, sidx1, didx0, didx1,
              rows0, rows1, isem0, isem1, gsem0, gsem1, ssem0, ssem1):
    c = lax.axis_index("c")
    s = lax.axis_index("s")
    sidx = (sidx0, sidx1)
    didx = (didx0, didx1)
    rows = (rows0, rows1)
    isem = (isem0, isem1)
    gsem = (gsem0, gsem1)
    ssem = (ssem0, ssem1)

    is0 = c == 0
    tabs = (jnp.where(is0, 0, 3), jnp.where(is0, 1, 4), jnp.where(is0, 2, 2))
    slots = (jnp.where(is0, 0, 3), jnp.where(is0, 1, 4), jnp.where(is0, 2, 5))
    bases = (0, 0, jnp.where(is0, 0, NBLK // 2))
    per_tiles = (NBLK // NS, NBLK // NS, (NBLK // 2) // NS)

    for tab, slot, base, per_tile in zip(tabs, slots, bases, per_tiles):
        ytab_t = ytab.at[tab]
        _fill_rows(rows0, 0.0)
        _zero_spmem(spmem, rows0, s)
        plsc.subcore_barrier()
        tile_b0 = base + s * per_tile

        def fetch_idx(blk, k):
            pltpu.async_copy(srcb.at[blk], sidx[k], isem[k])
            pltpu.async_copy(dstb.at[blk], didx[k], isem[k])

        def wait_idx(blk, k):
            pltpu.make_async_copy(srcb.at[blk], sidx[k], isem[k]).wait()
            pltpu.make_async_copy(dstb.at[blk], didx[k], isem[k]).wait()

        def fire_gathers(k):
            @pl.loop(0, NSTR)
            def _(j):
                pltpu.async_copy(ytab_t.at[sidx[k].at[j]],
                                 rows[k].at[pl.ds(j * IPS, IPS)], gsem[k])

        def drain_gathers(k):
            @pl.loop(0, NSTR)
            def _(j):
                pltpu.make_async_copy(ytab_t.at[sidx[k].at[j]],
                                      rows[k].at[pl.ds(j * IPS, IPS)],
                                      gsem[k]).wait()

        def fire_scatters(k):
            @pl.loop(0, NSTR)
            def _(j):
                pltpu.async_copy(rows[k].at[pl.ds(j * IPS, IPS)],
                                 spmem.at[didx[k].at[j]], ssem[k], add=True)

        def drain_scatters(k):
            @pl.loop(0, NSTR)
            def _(j):
                pltpu.make_async_copy(rows[k].at[pl.ds(j * IPS, IPS)],
                                      spmem.at[didx[k].at[j]], ssem[k]).wait()

        fetch_idx(tile_b0, 0)

        @pl.loop(0, per_tile, step=2)
        def _(bb):
            for k in (0, 1):
                b = bb + k
                blk = tile_b0 + b
                wait_idx(blk, k)
                fire_gathers(k)
                if k == 0:
                    @pl.when(bb > 0)
                    def _():
                        drain_scatters(1)
                    fetch_idx(blk + 1, 1)
                else:
                    drain_scatters(0)

                    @pl.when(bb + 2 < per_tile)
                    def _():
                        fetch_idx(blk + 1, 0)
                drain_gathers(k)
                fire_scatters(k)

        drain_scatters(1)
        plsc.subcore_barrier()
        pltpu.sync_copy(spmem.at[pl.ds(s * ROWS_PER_TILE, ROWS_PER_TILE)],
                        agg.at[slot].at[pl.ds(s * ROWS_PER_TILE, ROWS_PER_TILE)])
        plsc.subcore_barrier()


@jax.jit
def _agg_kernel(ytab, srcb, dstb):
    return pl.kernel(
        _agg_body,
        compiler_params=_SC_PARAMS,
        out_type=jax.ShapeDtypeStruct((6, NP8, LANES), jnp.float32),
        mesh=_sc_mesh(),
        scratch_types=[
            pltpu.VMEM_SHARED((NP8, LANES), jnp.float32),
            pltpu.VMEM((NSTR, IPS), jnp.int32),
            pltpu.VMEM((NSTR, IPS), jnp.int32),
            pltpu.VMEM((NSTR, IPS), jnp.int32),
            pltpu.VMEM((NSTR, IPS), jnp.int32),
            pltpu.VMEM((EB, LANES), jnp.float32),
            pltpu.VMEM((EB, LANES), jnp.float32),
            pltpu.SemaphoreType.DMA,
            pltpu.SemaphoreType.DMA,
            pltpu.SemaphoreType.DMA,
            pltpu.SemaphoreType.DMA,
            pltpu.SemaphoreType.DMA,
            pltpu.SemaphoreType.DMA,
        ],
    )(ytab, srcb, dstb)


RB = 2944
NBB = NP8 // RB
RBP = RB // 8
NROWS_P = NP8 // 8


def _prep_body(deg_ref, x_ref, y_ref, dinv_ref):
    degs = deg_ref[0] + deg_ref[1] + 1.0
    dinv = lax.rsqrt(degs)
    dinv_ref[...] = dinv
    x3 = x_ref[...]
    for t in range(CH):
        pieces = []
        for k in range(8):
            lo = t * LANES
            hi = min((t + 1) * LANES, DIN)
            xk = x3[:, k, lo:hi]
            if hi - lo < LANES:
                xk = jnp.concatenate(
                    [xk, jnp.zeros((RBP, LANES - (hi - lo)), jnp.float32)], axis=1)
            pieces.append(xk * dinv[:, k * LANES:k * LANES + 1])
        y_ref[t] = jnp.concatenate(pieces, axis=1)


@jax.jit
def _prep_kernel(degp_p, x3):
    return pl.pallas_call(
        _prep_body,
        grid=(NBB,),
        in_specs=[
            pl.BlockSpec((NC, RBP, 128), lambda i: (0, i, 0)),
            pl.BlockSpec((RBP, 8, DIN), lambda i: (i, 0, 0)),
        ],
        out_specs=[
            pl.BlockSpec((CH, RBP, 128), lambda i: (0, i, 0)),
            pl.BlockSpec((RBP, 128), lambda i: (i, 0)),
        ],
        out_shape=[
            jax.ShapeDtypeStruct((CH, NROWS_P, 128), jnp.float32),
            jax.ShapeDtypeStruct((NROWS_P, 128), jnp.float32),
        ],
    )(degp_p, x3)


RD = 1088
NBD = NP8 // RD
RDP = RD // 8


def _head_body(gf_ref, gl_ref, agg_ref, y_ref, dinv_ref, batch_ref,
               w1_ref, b1_ref, w2_ref, b2_ref, out_ref, pooled):
    i = pl.program_id(0)

    @pl.when(i == 0)
    def _():
        pooled[...] = jnp.zeros_like(pooled)

    packed = [agg_ref[0] + y_ref[0], agg_ref[1] + y_ref[1],
              agg_ref[2] + agg_ref[5] + y_ref[2],
              agg_ref[3] + y_ref[3], agg_ref[4] + y_ref[4]]
    dinv = dinv_ref[...]
    bat = batch_ref[...]
    pres = []
    bats = []
    for k in range(8):
        pre_k = jnp.concatenate(
            [p[:, k * LANES:(k + 1) * LANES] for p in packed], axis=1)
        pres.append(pre_k * dinv[:, k * LANES:k * LANES + 1])
        bats.append(bat[:, k:k + 1])
    pre = jnp.reshape(jnp.stack(pres, axis=1), (RD, DPAD))
    h = jnp.dot(pre, w1_ref[...], preferred_element_type=jnp.float32,
                precision=lax.Precision.HIGHEST)
    h = jnp.maximum(h + b1_ref[...], 0.0)

    rid = i * RD + lax.broadcasted_iota(jnp.int32, (RD, 1), 0)
    h = jnp.where(rid < N, h, 0.0)

    bcol = jnp.reshape(jnp.stack(bats, axis=1), (RD, 1))
    g0 = gf_ref[i]
    g1 = gl_ref[i]

    def upd(kk, _):
        g = g0 + kk
        m = bcol == g
        colmax = jnp.max(jnp.where(m, h, 0.0), axis=0)[None, None, :]
        pooled[pl.ds(g, 1)] = jnp.maximum(pooled[pl.ds(g, 1)], colmax)
        return 0

    lax.fori_loop(0, g1 - g0 + 1, upd, 0)

    @pl.when(i == NBD - 1)
    def _():
        pool2d = pooled[...][:, 0, :]
        o = jnp.dot(pool2d, w2_ref[...], preferred_element_type=jnp.float32,
                    precision=lax.Precision.HIGHEST)
        out_ref[...] = jnp.maximum(o + b2_ref[...], 0.0)


@jax.jit
def _head_kernel(gf, gl, agg, ytab, dinv, batchcol, w1p, b1r, w2, b2r):
    grid_spec = pltpu.PrefetchScalarGridSpec(
        num_scalar_prefetch=2,
        grid=(NBD,),
        in_specs=[
            pl.BlockSpec((6, RDP, 128), lambda i, *_: (0, i, 0)),
            pl.BlockSpec((CH, RDP, 128), lambda i, *_: (0, i, 0)),
            pl.BlockSpec((RDP, 128), lambda i, *_: (i, 0)),
            pl.BlockSpec((RDP, 8), lambda i, *_: (i, 0)),
            pl.BlockSpec((DPAD, DH), lambda i, *_: (0, 0)),
            pl.BlockSpec((1, DH), lambda i, *_: (0, 0)),
            pl.BlockSpec((DH, DE), lambda i, *_: (0, 0)),
            pl.BlockSpec((1, DE), lambda i, *_: (0, 0)),
        ],
        out_specs=pl.BlockSpec((G, DE), lambda i, *_: (0, 0)),
        scratch_shapes=[pltpu.VMEM((G, 1, DH), jnp.float32)],
    )
    return pl.pallas_call(
        _head_body,
        grid_spec=grid_spec,
        out_shape=jax.ShapeDtypeStruct((G, DE), jnp.float32),
    )(gf, gl, agg, ytab, dinv, batchcol, w1p, b1r, w2, b2r)


def kernel(x, edge_index, batch, W1, b1, W2, b2):
    src = edge_index[0].astype(jnp.int32).reshape(NBLK, NSTR, IPS)
    dst = edge_index[1].astype(jnp.int32).reshape(NBLK, NSTR, IPS)
    xp3 = jnp.pad(x, ((0, NP8 - N), (0, 0))).reshape(NROWS_P, 8, DIN)
    batch32 = jnp.pad(batch.astype(jnp.int32), (0, NP8 - N), mode="edge")
    batch_p = batch32.reshape(NROWS_P, 8)
    gf = batch32[::RD]
    gl = batch32[RD - 1::RD]
    w1p = jnp.concatenate([W1, jnp.zeros((DPAD - DIN, DH), jnp.float32)], axis=0)

    degp = _deg_kernel(dst)
    ytab_p, dinv_p = _prep_kernel(degp.reshape(NC, NROWS_P, 128), xp3)
    agg = _agg_kernel(ytab_p.reshape(CH, NP8, LANES), src, dst)
    return _head_kernel(gf, gl, agg.reshape(6, NROWS_P, 128), ytab_p, dinv_p,
                        batch_p, w1p, b1.reshape(1, DH), W2, b2.reshape(1, DE))
